# Initial kernel scaffold; baseline (speedup 1.0000x reference)
import jax
import jax.numpy as jnp
from jax.experimental import pallas as pl


def kernel(q, k, v, Wq, bq, Wk, bk, Wv, bv, Wo, bo):
    raise NotImplementedError("write your pallas kernel here")


# speedup vs baseline: 5.3160x; 5.3160x over previous
import math

import jax
import jax.numpy as jnp
import numpy as np
from jax.experimental import pallas as pl
from jax.experimental.pallas import tpu as pltpu

H = 16
D = 2048
DK = D // H
L = 2048
MT = 40
BM = 512
NBLK = L // BM
NEG = np.float32(-3.0e38)

_idx = np.asarray(jax.random.randint(jax.random.key(42), (L, MT), 0, L))
_C_np = np.zeros((L, L), np.float32)
np.add.at(_C_np, (np.arange(L)[:, None], _idx), 1.0)


def _dot(a, b, dims):
    return jax.lax.dot_general(a, b, (dims, ((), ())),
                               preferred_element_type=jnp.float32)


def _mkernel(q_ref, k_ref, wq_ref, wk_ref, bq_ref, bk_ref, c_ref,
             m_ref, sa_ref, kh_scr):
    h = pl.program_id(0)
    i = pl.program_id(1)

    @pl.when(i == 0)
    def _():
        bk_row = bk_ref[pl.ds(h, 1), :]
        kh_scr[:, :] = _dot(k_ref[:, :], wk_ref[:, :], ((1,), (1,))) + bk_row

    qb = q_ref[pl.ds(i * BM, BM), :]
    bq_row = bq_ref[pl.ds(h, 1), :]
    Qb = _dot(qb, wq_ref[:, :], ((1,), (1,))) + bq_row
    S = _dot(Qb, kh_scr[:, :], ((1,), (1,)))
    cb = c_ref[pl.ds(i * BM, BM), :]
    cf = cb.astype(jnp.float32)
    mx = jnp.max(jnp.where(cb > 0, S, NEG), axis=1)
    sm = jnp.sum(S * cf, axis=1) * np.float32(1.0 / L)
    m_ref[0, 0, :] = mx - sm
    kl = kh_scr[pl.ds(L - 1, 1), :]
    sa_ref[0, 0, :] = jnp.sum(Qb * kl, axis=1)


def _topk_kernel(m_ref, sa_ref, v_ref, vg_ref, s_ref, vs_ref):
    Mw = m_ref[:, :]
    iota = jax.lax.broadcasted_iota(jnp.int32, (H, L), 1)
    cols = []
    for _ in range(MT):
        mxv = jnp.max(Mw, axis=1, keepdims=True)
        cand = jnp.where(Mw >= mxv, iota, L)
        t = jnp.min(cand, axis=1, keepdims=True)
        cols.append(t)
        Mw = jnp.where(iota == t, NEG, Mw)
    ti = jnp.concatenate(cols, axis=1)

    iota3 = jax.lax.broadcasted_iota(jnp.int32, (H, MT, L), 2)
    oh3 = (iota3 == ti[:, :, None]).astype(jnp.float32)
    sa = sa_ref[:, :]
    s = jnp.sum(oh3 * sa[:, None, :], axis=2)
    s_ref[:, :] = jnp.pad(s, ((0, 0), (0, 128 - MT)))

    oh2 = jnp.reshape(oh3, (H * MT, L))
    vg_ref[:, :] = _dot(oh2, v_ref[:, :], ((1,), (0,)))
    vs_ref[:, :] = jnp.reshape(jnp.sum(v_ref[:, :], axis=0), (H, DK))


def _out_kernel(vg_ref, s_ref, vs_ref, wv_ref, bv_ref, wo_ref, bo_ref,
                out_ref, ctx_scr, rows_scr):
    j = pl.program_id(0)

    @pl.when(j == 0)
    def _():
        ctx_scr[:, :] = jnp.zeros((64, D), jnp.float32)
        sT = _dot(s_ref[:, :], jnp.eye(H, dtype=jnp.float32), ((0,), (0,)))
        vsum_row = vs_ref[:, :]
        for h in range(H):
            wv_h = wv_ref[h * DK:(h + 1) * DK, :]
            bv_h = bv_ref[h:h + 1, :]
            vsum_h = _dot(vsum_row, wv_h, ((1,), (1,))) + np.float32(L) * bv_h
            vc_h = _dot(vg_ref[h * MT:(h + 1) * MT, :], wv_h,
                        ((1,), (1,))) + bv_h
            s_h = sT[0:MT, h:h + 1]
            m = jnp.maximum(s_h, 0.0)
            e0 = jnp.exp(-m)
            es = jnp.exp(s_h - m)
            z = np.float32(L - 1) * e0 + es
            ctx_h = (e0 * (vsum_h - vc_h) + es * vc_h) / z
            ctx_scr[0:MT, h * DK:(h + 1) * DK] = ctx_h
            ctx_scr[MT:MT + 1, h * DK:(h + 1) * DK] = vsum_h * np.float32(1.0 / L)
        rows_scr[:, :] = _dot(ctx_scr[:, :], wo_ref[:, :],
                              ((1,), (1,))) + bo_ref[:, :]

    unif = rows_scr[MT:MT + 1, :]

    @pl.when(j == 0)
    def _():
        out_ref[:, :] = jnp.concatenate(
            [rows_scr[0:MT, :],
             jnp.broadcast_to(unif, (out_ref.shape[0] - MT, D))], axis=0)

    @pl.when(j > 0)
    def _():
        out_ref[:, :] = jnp.broadcast_to(unif, (out_ref.shape[0], D))


def kernel(q, k, v, Wq, bq, Wk, bk, Wv, bv, Wo, bo):
    q2 = q[0]
    k2 = k[0]
    v2 = v[0]
    c_const = jnp.asarray(_C_np, dtype=jnp.bfloat16)
    bq_r = bq.reshape(H, DK)
    bk_r = bk.reshape(H, DK)
    bv_r = bv.reshape(H, DK)
    bo_r = bo.reshape(1, D)

    m3, sa3 = pl.pallas_call(
        _mkernel,
        grid=(H, NBLK),
        in_specs=[
            pl.BlockSpec((L, D), lambda h, i: (0, 0)),
            pl.BlockSpec((L, D), lambda h, i: (0, 0)),
            pl.BlockSpec((DK, D), lambda h, i: (h, 0)),
            pl.BlockSpec((DK, D), lambda h, i: (h, 0)),
            pl.BlockSpec((H, DK), lambda h, i: (0, 0)),
            pl.BlockSpec((H, DK), lambda h, i: (0, 0)),
            pl.BlockSpec((L, L), lambda h, i: (0, 0)),
        ],
        out_specs=[
            pl.BlockSpec((1, 1, BM), lambda h, i: (h * NBLK + i, 0, 0)),
            pl.BlockSpec((1, 1, BM), lambda h, i: (h * NBLK + i, 0, 0)),
        ],
        out_shape=[
            jax.ShapeDtypeStruct((H * NBLK, 1, BM), jnp.float32),
            jax.ShapeDtypeStruct((H * NBLK, 1, BM), jnp.float32),
        ],
        scratch_shapes=[pltpu.VMEM((L, DK), jnp.float32)],
    )(q2, k2, Wq, Wk, bq_r, bk_r, c_const)

    m_arr = m3.reshape(H, L)
    sa_arr = sa3.reshape(H, L)

    vg, s_pad, vsum = pl.pallas_call(
        _topk_kernel,
        out_shape=[
            jax.ShapeDtypeStruct((H * MT, D), jnp.float32),
            jax.ShapeDtypeStruct((H, 128), jnp.float32),
            jax.ShapeDtypeStruct((H, DK), jnp.float32),
        ],
    )(m_arr, sa_arr, v2)

    vsum_row = vsum.reshape(1, D)

    out = pl.pallas_call(
        _out_kernel,
        grid=(8,),
        in_specs=[
            pl.BlockSpec((H * MT, D), lambda j: (0, 0)),
            pl.BlockSpec((H, 128), lambda j: (0, 0)),
            pl.BlockSpec((1, D), lambda j: (0, 0)),
            pl.BlockSpec((D, D), lambda j: (0, 0)),
            pl.BlockSpec((H, DK), lambda j: (0, 0)),
            pl.BlockSpec((D, D), lambda j: (0, 0)),
            pl.BlockSpec((1, D), lambda j: (0, 0)),
        ],
        out_specs=pl.BlockSpec((L // 8, D), lambda j: (j, 0)),
        out_shape=jax.ShapeDtypeStruct((L, D), jnp.float32),
        scratch_shapes=[pltpu.VMEM((64, D), jnp.float32),
                        pltpu.VMEM((64, D), jnp.float32)],
    )(vg, s_pad, vsum_row, Wv, bv_r, Wo, bo_r)

    return out[None]
